# baseline (speedup 1.0000x reference)
import dataclasses

import jax
import jax.numpy as jnp
from jax import lax
from jax.experimental import pallas as pl
from jax.experimental.pallas import tpu as pltpu
from jax.experimental.pallas import tpu_sc as plsc

N_USERS = 1000000
N_ITEMS = 100000
N_FACTORS = 32
BATCH = 16384

NUM_CORES = 2
NUM_SUBCORES = 16
NUM_WORKERS = NUM_CORES * NUM_SUBCORES
B_PER_W = BATCH // NUM_WORKERS
IDX_CHUNK = 128
CHUNKS_PER_W = B_PER_W // IDX_CHUNK
LANES = 16
ROWS_PER_WIDE = 4


def _body(users_hbm, items_hbm, u_hbm, v_hbm, out_hbm,
          uidx, vidx, urow, vrow, u_buf, v_buf, out_v, sems):
  wid = lax.axis_index("s") * NUM_CORES + lax.axis_index("c")

  pltpu.sync_copy(users_hbm.at[pl.ds(wid * CHUNKS_PER_W, CHUNKS_PER_W)], uidx)
  pltpu.sync_copy(items_hbm.at[pl.ds(wid * CHUNKS_PER_W, CHUNKS_PER_W)], vidx)

  for c in range(CHUNKS_PER_W):
    for i in range(IDX_CHUNK // LANES):
      s = pl.ds(i * LANES, LANES)
      urow.at[c][s] = uidx.at[c][s] >> 2
      vrow.at[c][s] = vidx.at[c][s] >> 2

  def fire(c):
    buf = c % 2
    return (pltpu.async_copy(u_hbm.at[urow.at[c]], u_buf.at[buf], sems[buf]),
            pltpu.async_copy(v_hbm.at[vrow.at[c]], v_buf.at[buf], sems[buf]))

  def compute(c):
    buf = c % 2
    @pl.loop(0, IDX_CHUNK, step=LANES)
    def _(j):
      rows = lax.iota(jnp.int32, LANES) + j
      uq = uidx.at[c][pl.ds(j, LANES)]
      vq = vidx.at[c][pl.ds(j, LANES)]
      off_u = (uq & (ROWS_PER_WIDE - 1)) << 5
      off_v = (vq & (ROWS_PER_WIDE - 1)) << 5
      acc = jnp.zeros((LANES,), jnp.float32)
      for k in range(N_FACTORS):
        uc = plsc.load_gather(u_buf.at[buf], [rows, off_u + k])
        vc = plsc.load_gather(v_buf.at[buf], [rows, off_v + k])
        acc = acc + uc * vc
      out_v[pl.ds(c * IDX_CHUNK + j, LANES)] = acc

  inflight = fire(0)
  for c in range(CHUNKS_PER_W):
    for cp in inflight:
      cp.wait()
    if c + 1 < CHUNKS_PER_W:
      inflight = fire(c + 1)
    compute(c)

  pltpu.sync_copy(out_v, out_hbm.at[pl.ds(wid * B_PER_W, B_PER_W)])


@jax.jit
def _pmf_sc(users, items, u_wide, v_wide):
  mesh = plsc.VectorSubcoreMesh(
      core_axis_name="c", subcore_axis_name="s",
      num_cores=NUM_CORES, num_subcores=NUM_SUBCORES)
  cp = pltpu.CompilerParams()
  if "needs_layout_passes" in pltpu.CompilerParams.__dataclass_fields__:
    cp = dataclasses.replace(cp, needs_layout_passes=False)
  run = pl.kernel(
      _body,
      compiler_params=cp,
      out_type=jax.ShapeDtypeStruct((BATCH,), jnp.float32),
      mesh=mesh,
      scratch_types=[
          pltpu.VMEM((CHUNKS_PER_W, IDX_CHUNK), jnp.int32),
          pltpu.VMEM((CHUNKS_PER_W, IDX_CHUNK), jnp.int32),
          pltpu.VMEM((CHUNKS_PER_W, IDX_CHUNK), jnp.int32),
          pltpu.VMEM((CHUNKS_PER_W, IDX_CHUNK), jnp.int32),
          pltpu.VMEM((2, IDX_CHUNK, 4 * N_FACTORS), jnp.float32),
          pltpu.VMEM((2, IDX_CHUNK, 4 * N_FACTORS), jnp.float32),
          pltpu.VMEM((B_PER_W,), jnp.float32),
          [pltpu.SemaphoreType.DMA, pltpu.SemaphoreType.DMA],
      ],
  )
  return run(users, items, u_wide, v_wide)


def kernel(users_index, items_index, U, V):
  users = users_index.astype(jnp.int32).reshape(BATCH // IDX_CHUNK, IDX_CHUNK)
  items = items_index.astype(jnp.int32).reshape(BATCH // IDX_CHUNK, IDX_CHUNK)
  u_wide = U.reshape(N_USERS // ROWS_PER_WIDE, ROWS_PER_WIDE * N_FACTORS)
  v_wide = V.reshape(N_ITEMS // ROWS_PER_WIDE, ROWS_PER_WIDE * N_FACTORS)
  return _pmf_sc(users, items, u_wide, v_wide)
